# baseline (speedup 1.0000x reference)
import jax
import jax.numpy as jnp
from jax import lax
from jax.experimental import pallas as pl
from jax.experimental.pallas import tpu as pltpu
from jax.experimental.pallas import tpu_sc as plsc

F32 = jnp.float32
I32 = jnp.int32


def _router_body(x_ref, wgt_ref, bias_ref, Wgs_ref, Wus_ref, Wds_ref,
                 comb_ref, act_ref, shared_ref):
    xb = x_ref[...]
    g = jax.nn.silu(jnp.dot(xb, Wgs_ref[0], preferred_element_type=F32))
    u = jnp.dot(xb, Wus_ref[0], preferred_element_type=F32)
    shared_ref[...] = jnp.dot(g * u, Wds_ref[0], preferred_element_type=F32)

    logits = lax.dot_general(
        xb, wgt_ref[...], (((1,), (1,)), ((), ())),
        preferred_element_type=F32) + bias_ref[...]
    m = jnp.max(logits, axis=1, keepdims=True)
    p = jnp.exp(logits - m)
    p = p / jnp.sum(p, axis=1, keepdims=True)
    idx = lax.broadcasted_iota(I32, p.shape, 1)
    big = jnp.int32(1 << 30)
    m1 = jnp.max(p, axis=1, keepdims=True)
    i1 = jnp.min(jnp.where(p == m1, idx, big), axis=1, keepdims=True)
    mask1 = idx == i1
    p2 = jnp.where(mask1, -jnp.inf, p)
    m2 = jnp.max(p2, axis=1, keepdims=True)
    i2 = jnp.min(jnp.where(p2 == m2, idx, big), axis=1, keepdims=True)
    mask2 = idx == i2
    comb = (jnp.where(mask1, m1, 0.0) + jnp.where(mask2, m2, 0.0)) / (m1 + m2)
    comb_ref[...] = comb
    counts = jnp.sum(jnp.where(comb > 0, 1.0, 0.0), axis=0, keepdims=True)
    act_ref[...] = (counts > 0).astype(I32)


def _router(x_flat, W_gate, bias2d, Wg_s, Wu_s, Wd_s):
    N, C = x_flat.shape
    E = W_gate.shape[0]
    return pl.pallas_call(
        _router_body,
        out_shape=(jax.ShapeDtypeStruct((N, E), F32),
                   jax.ShapeDtypeStruct((1, E), I32),
                   jax.ShapeDtypeStruct((N, C), F32)),
    )(x_flat, W_gate, bias2d, Wg_s, Wu_s, Wd_s)


def _make_sched_kernel(E):
    mesh = plsc.VectorSubcoreMesh(core_axis_name="c", subcore_axis_name="s")

    def body(act_hbm, sched_hbm, act_v, sched_v):
        wid = lax.axis_index("s") * 2 + lax.axis_index("c")

        @pl.when(wid == 0)
        def _():
            pltpu.sync_copy(act_hbm, act_v.at[pl.ds(0, E)])

            def step(e, n):
                a = act_v[pl.ds(e, 16)][0]
                idx = jnp.where(a > 0, n, E + 32)
                sched_v[pl.ds(idx, 16)] = jnp.full((16,), e, I32)
                return n + jnp.where(a > 0, 1, 0)

            n = lax.fori_loop(0, E, step, jnp.int32(0))
            sched_v[pl.ds(E, 16)] = jnp.full((16,), n, I32)
            pltpu.sync_copy(sched_v.at[pl.ds(0, E + 16)], sched_hbm)

    return pl.kernel(
        body,
        out_type=jax.ShapeDtypeStruct((E + 16,), I32),
        mesh=mesh,
        scratch_types=[pltpu.VMEM((E + 16,), I32), pltpu.VMEM((E + 48,), I32)],
    )


def _moe_body(s_ref, x_ref, comb_ref, shared_ref, Wga_ref, Wgb_ref,
              Wua_ref, Wub_ref, Wda_ref, Wdb_ref, out_ref):
    t = pl.program_id(0)
    n = s_ref[s_ref.shape[0] - 16]
    ae = s_ref[jnp.minimum(t, n - 1)]
    xb = x_ref[...]

    @pl.when(t == 0)
    def _init():
        out_ref[...] = shared_ref[...]

    @pl.when(t < n)
    def _routed():
        h = xb.shape[1] // 2
        xa = xb[:, :h]
        xc = xb[:, h:]
        g = jax.nn.silu(jnp.dot(xa, Wga_ref[0], preferred_element_type=F32)
                        + jnp.dot(xc, Wgb_ref[0], preferred_element_type=F32))
        u = (jnp.dot(xa, Wua_ref[0], preferred_element_type=F32)
             + jnp.dot(xc, Wub_ref[0], preferred_element_type=F32))
        hu = g * u
        hf = hu.shape[1] // 2
        contrib = (jnp.dot(hu[:, :hf], Wda_ref[0], preferred_element_type=F32)
                   + jnp.dot(hu[:, hf:], Wdb_ref[0],
                             preferred_element_type=F32))
        eids = lax.broadcasted_iota(I32, comb_ref.shape, 1)
        col = jnp.sum(jnp.where(eids == ae, comb_ref[...], 0.0),
                      axis=1, keepdims=True)
        out_ref[...] += col * contrib


def kernel(x, W_gate, expert_bias, Wg, Wu, Wd, Wg_s, Wu_s, Wd_s):
    Bq, Tq, C = x.shape
    N = Bq * Tq
    E, _, FF = Wg.shape
    x_flat = x.reshape(N, C)
    bias2d = expert_bias.reshape(1, E)

    comb, act, shared = _router(x_flat, W_gate, bias2d, Wg_s, Wu_s, Wd_s)
    sched = _make_sched_kernel(E)(act.reshape(E))

    def _slot(t, s):
        return s[jnp.minimum(t, s[E] - 1)]

    grid_spec = pltpu.PrefetchScalarGridSpec(
        num_scalar_prefetch=1,
        grid=(E,),
        in_specs=[
            pl.BlockSpec((N, C), lambda t, s: (0, 0)),
            pl.BlockSpec((N, E), lambda t, s: (0, 0)),
            pl.BlockSpec((N, C), lambda t, s: (0, 0)),
            pl.BlockSpec((1, C // 2, FF), lambda t, s: (_slot(t, s), 0, 0)),
            pl.BlockSpec((1, C // 2, FF), lambda t, s: (_slot(t, s), 1, 0)),
            pl.BlockSpec((1, C // 2, FF), lambda t, s: (_slot(t, s), 0, 0)),
            pl.BlockSpec((1, C // 2, FF), lambda t, s: (_slot(t, s), 1, 0)),
            pl.BlockSpec((1, FF // 2, C), lambda t, s: (_slot(t, s), 0, 0)),
            pl.BlockSpec((1, FF // 2, C), lambda t, s: (_slot(t, s), 1, 0)),
        ],
        out_specs=pl.BlockSpec((N, C), lambda t, s: (0, 0)),
    )
    out = pl.pallas_call(
        _moe_body,
        grid_spec=grid_spec,
        out_shape=jax.ShapeDtypeStruct((N, C), F32),
        compiler_params=pltpu.CompilerParams(
            dimension_semantics=("arbitrary",)),
    )(sched, x_flat, comb, shared, Wg, Wg, Wu, Wu, Wd, Wd)
    return out.reshape(Bq, Tq, C)


# speedup vs baseline: 1.0072x; 1.0072x over previous
import jax
import jax.numpy as jnp
from jax import lax
from jax.experimental import pallas as pl
from jax.experimental.pallas import tpu as pltpu
from jax.experimental.pallas import tpu_sc as plsc

F32 = jnp.float32
I32 = jnp.int32


def _router_body(x_ref, wgt_ref, bias_ref, Wgs_ref, Wus_ref, Wds_ref,
                 comb_ref, act_ref, shared_ref):
    xb = x_ref[...]
    g = jax.nn.silu(jnp.dot(xb, Wgs_ref[0], preferred_element_type=F32))
    u = jnp.dot(xb, Wus_ref[0], preferred_element_type=F32)
    shared_ref[...] = jnp.dot(g * u, Wds_ref[0], preferred_element_type=F32)

    logits = lax.dot_general(
        xb, wgt_ref[...], (((1,), (1,)), ((), ())),
        preferred_element_type=F32) + bias_ref[...]
    m = jnp.max(logits, axis=1, keepdims=True)
    p = jnp.exp(logits - m)
    p = p / jnp.sum(p, axis=1, keepdims=True)
    idx = lax.broadcasted_iota(I32, p.shape, 1)
    big = jnp.int32(1 << 30)
    m1 = jnp.max(p, axis=1, keepdims=True)
    i1 = jnp.min(jnp.where(p == m1, idx, big), axis=1, keepdims=True)
    mask1 = idx == i1
    p2 = jnp.where(mask1, -jnp.inf, p)
    m2 = jnp.max(p2, axis=1, keepdims=True)
    i2 = jnp.min(jnp.where(p2 == m2, idx, big), axis=1, keepdims=True)
    mask2 = idx == i2
    comb = (jnp.where(mask1, m1, 0.0) + jnp.where(mask2, m2, 0.0)) / (m1 + m2)
    comb_ref[...] = comb
    counts = jnp.sum(jnp.where(comb > 0, 1.0, 0.0), axis=0, keepdims=True)
    act_ref[...] = (counts > 0).astype(I32)


def _router(x_flat, W_gate, bias2d, Wg_s, Wu_s, Wd_s):
    N, C = x_flat.shape
    E = W_gate.shape[0]
    return pl.pallas_call(
        _router_body,
        out_shape=(jax.ShapeDtypeStruct((N, E), F32),
                   jax.ShapeDtypeStruct((1, E), I32),
                   jax.ShapeDtypeStruct((N, C), F32)),
    )(x_flat, W_gate, bias2d, Wg_s, Wu_s, Wd_s)


def _make_sched_kernel(E):
    mesh = plsc.VectorSubcoreMesh(core_axis_name="c", subcore_axis_name="s")

    def body(act_hbm, sched_hbm, act_v, sched_v):
        wid = lax.axis_index("s") * 2 + lax.axis_index("c")

        @pl.when(wid == 0)
        def _():
            pltpu.sync_copy(act_hbm, act_v.at[pl.ds(0, E)])

            def step(e, n):
                a = act_v[pl.ds(e, 16)][0]
                idx = jnp.where(a > 0, n, E + 32)
                sched_v[pl.ds(idx, 16)] = jnp.full((16,), e, I32)
                return n + jnp.where(a > 0, 1, 0)

            n = lax.fori_loop(0, E, step, jnp.int32(0))
            sched_v[pl.ds(E, 16)] = jnp.full((16,), n, I32)
            pltpu.sync_copy(sched_v.at[pl.ds(0, E + 16)], sched_hbm)

    return pl.kernel(
        body,
        out_type=jax.ShapeDtypeStruct((E + 16,), I32),
        mesh=mesh,
        scratch_types=[pltpu.VMEM((E + 16,), I32), pltpu.VMEM((E + 48,), I32)],
    )


def _moe_body(s_ref, x_ref, comb_ref, shared_ref, Wg_ref, Wu_ref, Wd_ref,
              out_ref):
    t = pl.program_id(0)
    n = s_ref[s_ref.shape[0] - 16]
    ae = s_ref[jnp.minimum(t, n - 1)]
    xb = x_ref[...]

    @pl.when(t == 0)
    def _init():
        out_ref[...] = shared_ref[...]

    @pl.when(t < n)
    def _routed():
        g = jax.nn.silu(jnp.dot(xb, Wg_ref[0], preferred_element_type=F32))
        u = jnp.dot(xb, Wu_ref[0], preferred_element_type=F32)
        contrib = jnp.dot(g * u, Wd_ref[0], preferred_element_type=F32)
        eids = lax.broadcasted_iota(I32, comb_ref.shape, 1)
        col = jnp.sum(jnp.where(eids == ae, comb_ref[...], 0.0),
                      axis=1, keepdims=True)
        out_ref[...] += col * contrib


def kernel(x, W_gate, expert_bias, Wg, Wu, Wd, Wg_s, Wu_s, Wd_s):
    Bq, Tq, C = x.shape
    N = Bq * Tq
    E, _, FF = Wg.shape
    x_flat = x.reshape(N, C)
    bias2d = expert_bias.reshape(1, E)

    comb, act, shared = _router(x_flat, W_gate, bias2d, Wg_s, Wu_s, Wd_s)
    sched = _make_sched_kernel(E)(act.reshape(E))

    def _slot(t, s):
        return s[jnp.minimum(t, s[E] - 1)]

    grid_spec = pltpu.PrefetchScalarGridSpec(
        num_scalar_prefetch=1,
        grid=(E,),
        in_specs=[
            pl.BlockSpec((N, C), lambda t, s: (0, 0)),
            pl.BlockSpec((N, E), lambda t, s: (0, 0)),
            pl.BlockSpec((N, C), lambda t, s: (0, 0)),
            pl.BlockSpec((1, C, FF), lambda t, s: (_slot(t, s), 0, 0)),
            pl.BlockSpec((1, C, FF), lambda t, s: (_slot(t, s), 0, 0)),
            pl.BlockSpec((1, FF, C), lambda t, s: (_slot(t, s), 0, 0)),
        ],
        out_specs=pl.BlockSpec((N, C), lambda t, s: (0, 0)),
    )
    out = pl.pallas_call(
        _moe_body,
        grid_spec=grid_spec,
        out_shape=jax.ShapeDtypeStruct((N, C), F32),
        compiler_params=pltpu.CompilerParams(
            dimension_semantics=("arbitrary",)),
    )(sched, x_flat, comb, shared, Wg, Wu, Wd)
    return out.reshape(Bq, Tq, C)


# speedup vs baseline: 1.0124x; 1.0051x over previous
import jax
import jax.numpy as jnp
from jax import lax
from jax.experimental import pallas as pl
from jax.experimental.pallas import tpu as pltpu
from jax.experimental.pallas import tpu_sc as plsc

F32 = jnp.float32
I32 = jnp.int32


def _shared_body(x_ref, Wgs_ref, Wus_ref, Wds_ref, shared_ref):
    xb = x_ref[...]
    g = jax.nn.silu(jnp.dot(xb, Wgs_ref[0], preferred_element_type=F32))
    u = jnp.dot(xb, Wus_ref[0], preferred_element_type=F32)
    shared_ref[...] = jnp.dot(g * u, Wds_ref[0], preferred_element_type=F32)


def _router_body(x_ref, wgt_ref, bias_ref, comb_ref, act_ref):
    xb = x_ref[...]
    logits = lax.dot_general(
        xb, wgt_ref[...], (((1,), (1,)), ((), ())),
        preferred_element_type=F32) + bias_ref[...]
    m = jnp.max(logits, axis=1, keepdims=True)
    p = jnp.exp(logits - m)
    p = p / jnp.sum(p, axis=1, keepdims=True)
    idx = lax.broadcasted_iota(I32, p.shape, 1)
    big = jnp.int32(1 << 30)
    m1 = jnp.max(p, axis=1, keepdims=True)
    i1 = jnp.min(jnp.where(p == m1, idx, big), axis=1, keepdims=True)
    mask1 = idx == i1
    p2 = jnp.where(mask1, -jnp.inf, p)
    m2 = jnp.max(p2, axis=1, keepdims=True)
    i2 = jnp.min(jnp.where(p2 == m2, idx, big), axis=1, keepdims=True)
    mask2 = idx == i2
    comb = (jnp.where(mask1, m1, 0.0) + jnp.where(mask2, m2, 0.0)) / (m1 + m2)
    comb_ref[...] = comb
    counts = jnp.sum(jnp.where(comb > 0, 1.0, 0.0), axis=0, keepdims=True)
    act_ref[...] = (counts > 0).astype(I32)


def _router(x_flat, W_gate, bias2d):
    N, _ = x_flat.shape
    E = W_gate.shape[0]
    return pl.pallas_call(
        _router_body,
        out_shape=(jax.ShapeDtypeStruct((N, E), F32),
                   jax.ShapeDtypeStruct((1, E), I32)),
    )(x_flat, W_gate, bias2d)


def _shared(x_flat, Wg_s, Wu_s, Wd_s):
    N, C = x_flat.shape
    return pl.pallas_call(
        _shared_body,
        out_shape=jax.ShapeDtypeStruct((N, C), F32),
    )(x_flat, Wg_s, Wu_s, Wd_s)


def _make_sched_kernel(E):
    mesh = plsc.VectorSubcoreMesh(core_axis_name="c", subcore_axis_name="s")

    def body(act_hbm, sched_hbm, act_v, sched_v):
        wid = lax.axis_index("s") * 2 + lax.axis_index("c")

        @pl.when(wid == 0)
        def _():
            pltpu.sync_copy(act_hbm, act_v.at[pl.ds(0, E)])

            def step(e, n):
                a = act_v[pl.ds(e, 16)][0]
                idx = jnp.where(a > 0, n, E + 32)
                sched_v[pl.ds(idx, 16)] = jnp.full((16,), e, I32)
                return n + jnp.where(a > 0, 1, 0)

            n = lax.fori_loop(0, E, step, jnp.int32(0))
            sched_v[pl.ds(E, 16)] = jnp.full((16,), n, I32)
            pltpu.sync_copy(sched_v.at[pl.ds(0, E + 16)], sched_hbm)

    return pl.kernel(
        body,
        out_type=jax.ShapeDtypeStruct((E + 16,), I32),
        mesh=mesh,
        scratch_types=[pltpu.VMEM((E + 16,), I32), pltpu.VMEM((E + 48,), I32)],
    )


def _moe_body(s_ref, x_ref, comb_ref, shared_ref, Wg_ref, Wu_ref, Wd_ref,
              out_ref):
    t = pl.program_id(0)
    n = s_ref[s_ref.shape[0] - 16]
    ae = s_ref[jnp.minimum(t, n - 1)]
    xb = x_ref[...]

    @pl.when(t == 0)
    def _init():
        out_ref[...] = shared_ref[...]

    @pl.when(t < n)
    def _routed():
        g = jax.nn.silu(jnp.dot(xb, Wg_ref[0], preferred_element_type=F32))
        u = jnp.dot(xb, Wu_ref[0], preferred_element_type=F32)
        contrib = jnp.dot(g * u, Wd_ref[0], preferred_element_type=F32)
        eids = lax.broadcasted_iota(I32, comb_ref.shape, 1)
        col = jnp.sum(jnp.where(eids == ae, comb_ref[...], 0.0),
                      axis=1, keepdims=True)
        out_ref[...] += col * contrib


def kernel(x, W_gate, expert_bias, Wg, Wu, Wd, Wg_s, Wu_s, Wd_s):
    Bq, Tq, C = x.shape
    N = Bq * Tq
    E, _, FF = Wg.shape
    x_flat = x.reshape(N, C)
    bias2d = expert_bias.reshape(1, E)

    comb, act = _router(x_flat, W_gate, bias2d)
    sched = _make_sched_kernel(E)(act.reshape(E))
    shared = _shared(x_flat, Wg_s, Wu_s, Wd_s)

    def _slot(t, s):
        return s[jnp.minimum(t, s[E] - 1)]

    grid_spec = pltpu.PrefetchScalarGridSpec(
        num_scalar_prefetch=1,
        grid=(E,),
        in_specs=[
            pl.BlockSpec((N, C), lambda t, s: (0, 0)),
            pl.BlockSpec((N, E), lambda t, s: (0, 0)),
            pl.BlockSpec((N, C), lambda t, s: (0, 0)),
            pl.BlockSpec((1, C, FF), lambda t, s: (_slot(t, s), 0, 0)),
            pl.BlockSpec((1, C, FF), lambda t, s: (_slot(t, s), 0, 0)),
            pl.BlockSpec((1, FF, C), lambda t, s: (_slot(t, s), 0, 0)),
        ],
        out_specs=pl.BlockSpec((N, C), lambda t, s: (0, 0)),
    )
    out = pl.pallas_call(
        _moe_body,
        grid_spec=grid_spec,
        out_shape=jax.ShapeDtypeStruct((N, C), F32),
        compiler_params=pltpu.CompilerParams(
            dimension_semantics=("arbitrary",)),
    )(sched, x_flat, comb, shared, Wg, Wu, Wd)
    return out.reshape(Bq, Tq, C)
